# baseline (speedup 1.0000x reference)
import functools

import jax
import jax.numpy as jnp
from jax.experimental import pallas as pl
from jax.experimental.pallas import tpu as pltpu


def _pick_block(n, target):
    if n % target == 0:
        return target
    for b in range(min(target, n), 0, -1):
        if n % b == 0:
            return b
    return n


def _swap_halves(x):
    return jnp.concatenate([x[:, 64:], x[:, :64]], axis=1)


def _edge_agg_body(src_ref, dst_ref, rel_ref, ent2_ref, reldup_ref, agg2_ref,
                   deg_ref, *, block_e):
    @pl.when(pl.program_id(0) == 0)
    def _init():
        agg2_ref[...] = jnp.zeros(agg2_ref.shape, agg2_ref.dtype)
        deg_ref[...] = jnp.zeros(deg_ref.shape, deg_ref.dtype)

    lane = jax.lax.broadcasted_iota(jnp.int32, (1, 128), 1)

    def body(j, carry):
        s = src_ref[0, 0, j]
        d = dst_ref[0, 0, j]
        r = rel_ref[0, 0, j]
        msg = ent2_ref[pl.ds(s // 2, 1), :] * reldup_ref[pl.ds(r, 1), :]
        sel = jnp.where(s % 2 == d % 2, msg, _swap_halves(msg))
        contrib = jnp.where(lane // 64 == d % 2, sel, 0.0)
        agg2_ref[pl.ds(d // 2, 1), :] = agg2_ref[pl.ds(d // 2, 1), :] + contrib
        dcount = jnp.where(lane == d % 128, 1.0, 0.0)
        deg_ref[pl.ds(d // 128, 1), :] = deg_ref[pl.ds(d // 128, 1), :] + dcount
        return carry

    jax.lax.fori_loop(0, block_e, body, 0)


def _user_agg_body(rows_ref, cols_ref, vals_ref, ent2_ref, uagg2_ref, *,
                   block_e):
    @pl.when(pl.program_id(0) == 0)
    def _init():
        uagg2_ref[...] = jnp.zeros(uagg2_ref.shape, uagg2_ref.dtype)

    lane = jax.lax.broadcasted_iota(jnp.int32, (1, 128), 1)

    def body(j, carry):
        rw = rows_ref[0, 0, j]
        c = cols_ref[0, 0, j]
        v = vals_ref[0, 0, j]
        row = ent2_ref[pl.ds(c // 2, 1), :] * v
        sel = jnp.where(c % 2 == rw % 2, row, _swap_halves(row))
        contrib = jnp.where(lane // 64 == rw % 2, sel, 0.0)
        uagg2_ref[pl.ds(rw // 2, 1), :] = (
            uagg2_ref[pl.ds(rw // 2, 1), :] + contrib)
        return carry

    jax.lax.fori_loop(0, block_e, body, 0)


def _small_body(latent_ref, relemb_ref, att_ref, m_ref, cor_ref, *, temp):
    att = att_ref[...]
    mx = jnp.max(att, axis=-1, keepdims=True)
    e = jnp.exp(att - mx)
    sm = e / jnp.sum(e, axis=-1, keepdims=True)
    disen = jax.lax.dot_general(
        sm, relemb_ref[...], (((1,), (0,)), ((), ())),
        preferred_element_type=jnp.float32)
    m_ref[...] = jax.lax.dot_general(
        latent_ref[...], disen, (((0,), (0,)), ((), ())),
        preferred_element_type=jnp.float32)
    nrm = jnp.sqrt(jnp.sum(att * att, axis=-1, keepdims=True))
    natt = att / jnp.maximum(nrm, 1e-12)
    logits = jax.lax.dot_general(
        natt, natt, (((1,), (1,)), ((), ())),
        preferred_element_type=jnp.float32) / temp
    lmx = jnp.max(logits, axis=-1)
    lse = lmx + jnp.log(jnp.sum(jnp.exp(logits - lmx[:, None]), axis=-1))
    f = logits.shape[0]
    eye = (jax.lax.broadcasted_iota(jnp.int32, (f, f), 0) ==
           jax.lax.broadcasted_iota(jnp.int32, (f, f), 1))
    trace = jnp.sum(jnp.where(eye, logits, 0.0))
    cor_ref[...] = jnp.reshape(jnp.sum(lse) - trace, (1, 1))


def _ent_epilogue_body(agg_ref, deg_ref, resin_ref, new_ref, resout_ref):
    x = agg_ref[...] / jnp.maximum(deg_ref[...], 1.0)
    n = jnp.sqrt(jnp.sum(x * x, axis=-1, keepdims=True))
    y = x / jnp.maximum(n, 1e-12)
    new_ref[...] = y
    resout_ref[...] = resin_ref[...] + y


def _user_epilogue_body(uagg_ref, usr_ref, m_ref, resin_ref, new_ref,
                        resout_ref):
    u = uagg_ref[...]
    factor = jax.lax.dot_general(
        usr_ref[...], m_ref[...], (((1,), (0,)), ((), ())),
        preferred_element_type=jnp.float32)
    x = u * factor + u
    n = jnp.sqrt(jnp.sum(x * x, axis=-1, keepdims=True))
    y = x / jnp.maximum(n, 1e-12)
    new_ref[...] = y
    resout_ref[...] = resin_ref[...] + y


def kernel(edge_index, edge_rel, im_rows, im_cols, im_vals, user_emb,
           entity_emb, latent_emb, rel_emb, disen_weight_att):
    num_edges = edge_rel.shape[0]
    nnz = im_rows.shape[0]
    num_users, dim = user_emb.shape
    num_entities = entity_emb.shape[0]
    temp = 0.2

    eb = _pick_block(num_edges, 4000)
    ge = num_edges // eb
    ib = _pick_block(nnz, 4000)
    gi = nnz // ib

    src3 = edge_index[0].reshape(ge, 1, eb)
    dst3 = edge_index[1].reshape(ge, 1, eb)
    rel3 = edge_rel.reshape(ge, 1, eb)
    rows3 = im_rows.reshape(gi, 1, ib)
    cols3 = im_cols.reshape(gi, 1, ib)
    vals3 = im_vals.reshape(gi, 1, ib)

    ent_pack = num_entities // 2
    usr_pack = num_users // 2
    deg_rows = -(-num_entities // 128)
    rel_dup = jnp.concatenate([rel_emb, rel_emb], axis=1)

    idx_spec = lambda b: pl.BlockSpec((1, 1, b), lambda i: (i, 0, 0),
                                      memory_space=pltpu.SMEM)
    full = lambda shape: pl.BlockSpec(shape, lambda i: tuple(0 for _ in shape))

    edge_call = pl.pallas_call(
        functools.partial(_edge_agg_body, block_e=eb),
        grid=(ge,),
        in_specs=[idx_spec(eb), idx_spec(eb), idx_spec(eb),
                  full((ent_pack, 128)), full(rel_dup.shape)],
        out_specs=[full((ent_pack, 128)), full((deg_rows, 128))],
        out_shape=[jax.ShapeDtypeStruct((ent_pack, 128), jnp.float32),
                   jax.ShapeDtypeStruct((deg_rows, 128), jnp.float32)],
    )

    user_call = pl.pallas_call(
        functools.partial(_user_agg_body, block_e=ib),
        grid=(gi,),
        in_specs=[idx_spec(ib), idx_spec(ib), idx_spec(ib),
                  full((ent_pack, 128))],
        out_specs=[full((usr_pack, 128))],
        out_shape=[jax.ShapeDtypeStruct((usr_pack, 128), jnp.float32)],
    )

    m_mat, cor2 = pl.pallas_call(
        functools.partial(_small_body, temp=temp),
        in_specs=[pl.BlockSpec(latent_emb.shape, lambda: (0, 0)),
                  pl.BlockSpec(rel_emb.shape, lambda: (0, 0)),
                  pl.BlockSpec(disen_weight_att.shape, lambda: (0, 0))],
        out_specs=[pl.BlockSpec((dim, dim), lambda: (0, 0)),
                   pl.BlockSpec((1, 1), lambda: (0, 0))],
        out_shape=[jax.ShapeDtypeStruct((dim, dim), jnp.float32),
                   jax.ShapeDtypeStruct((1, 1), jnp.float32)],
    )(latent_emb, rel_emb, disen_weight_att)

    entb = _pick_block(num_entities, 2000)
    gent = num_entities // entb
    ent_epi = pl.pallas_call(
        _ent_epilogue_body,
        grid=(gent,),
        in_specs=[pl.BlockSpec((entb, dim), lambda i: (i, 0)),
                  pl.BlockSpec((entb, 1), lambda i: (i, 0)),
                  pl.BlockSpec((entb, dim), lambda i: (i, 0))],
        out_specs=[pl.BlockSpec((entb, dim), lambda i: (i, 0)),
                   pl.BlockSpec((entb, dim), lambda i: (i, 0))],
        out_shape=[jax.ShapeDtypeStruct((num_entities, dim), jnp.float32),
                   jax.ShapeDtypeStruct((num_entities, dim), jnp.float32)],
    )

    usrb = _pick_block(num_users, 2000)
    gusr = num_users // usrb
    usr_epi = pl.pallas_call(
        _user_epilogue_body,
        grid=(gusr,),
        in_specs=[pl.BlockSpec((usrb, dim), lambda i: (i, 0)),
                  pl.BlockSpec((usrb, dim), lambda i: (i, 0)),
                  pl.BlockSpec((dim, dim), lambda i: (0, 0)),
                  pl.BlockSpec((usrb, dim), lambda i: (i, 0))],
        out_specs=[pl.BlockSpec((usrb, dim), lambda i: (i, 0)),
                   pl.BlockSpec((usrb, dim), lambda i: (i, 0))],
        out_shape=[jax.ShapeDtypeStruct((num_users, dim), jnp.float32),
                   jax.ShapeDtypeStruct((num_users, dim), jnp.float32)],
    )

    ent = entity_emb
    usr = user_emb
    eres = entity_emb
    ures = user_emb
    for _ in range(2):
        ent2 = ent.reshape(ent_pack, 128)
        agg2, deg2 = edge_call(src3, dst3, rel3, ent2, rel_dup)
        agg = agg2.reshape(num_entities, dim)
        deg = deg2.reshape(deg_rows * 128)[:num_entities, None]
        (uagg2,) = user_call(rows3, cols3, vals3, ent2)
        uagg = uagg2.reshape(num_users, dim)
        ent, eres = ent_epi(agg, deg, eres)
        usr, ures = usr_epi(uagg, usr, m_mat, ures)

    return eres, ures, cor2.reshape(())


# speedup vs baseline: 1.2666x; 1.2666x over previous
import functools

import jax
import jax.numpy as jnp
from jax.experimental import pallas as pl
from jax.experimental.pallas import tpu as pltpu


def _pick_block(n, target):
    if n % target == 0:
        return target
    for b in range(min(target, n), 0, -1):
        if n % b == 0:
            return b
    return n


def _swap_halves(x):
    return jnp.concatenate([x[:, 64:], x[:, :64]], axis=1)


def _edge_agg_body(src_ref, dst_ref, rel_ref, ent2_ref, reldup_ref, agg2_ref,
                   *, block_e):
    @pl.when(pl.program_id(0) == 0)
    def _init():
        agg2_ref[...] = jnp.zeros(agg2_ref.shape, agg2_ref.dtype)

    lane = jax.lax.broadcasted_iota(jnp.int32, (1, 128), 1)

    def body(j, carry):
        s = src_ref[0, 0, j]
        d = dst_ref[0, 0, j]
        r = rel_ref[0, 0, j]
        msg = ent2_ref[pl.ds(s // 2, 1), :] * reldup_ref[pl.ds(r, 1), :]
        sel = jnp.where(s % 2 == d % 2, msg, _swap_halves(msg))
        contrib = jnp.where(lane // 64 == d % 2, sel, 0.0)
        agg2_ref[pl.ds(d // 2, 1), :] = agg2_ref[pl.ds(d // 2, 1), :] + contrib
        return carry

    jax.lax.fori_loop(0, block_e, body, 0)


def _deg_body(dst_ref, dega_ref, degb_ref, *, block_e):
    @pl.when(pl.program_id(0) == 0)
    def _init():
        dega_ref[...] = jnp.zeros(dega_ref.shape, dega_ref.dtype)
        degb_ref[...] = jnp.zeros(degb_ref.shape, degb_ref.dtype)

    lane = jax.lax.broadcasted_iota(jnp.int32, (1, 128), 1)
    half = block_e // 2

    def body(j, carry):
        da = dst_ref[0, 0, j]
        db = dst_ref[0, 0, j + half]
        ca = jnp.where(lane == da % 128, 1.0, 0.0)
        cb = jnp.where(lane == db % 128, 1.0, 0.0)
        dega_ref[pl.ds(da // 128, 1), :] = dega_ref[pl.ds(da // 128, 1), :] + ca
        degb_ref[pl.ds(db // 128, 1), :] = degb_ref[pl.ds(db // 128, 1), :] + cb
        return carry

    jax.lax.fori_loop(0, half, body, 0)


def _user_agg_body(rows_ref, cols_ref, vals_ref, ent2_ref, ua_ref, ub_ref, *,
                   block_e):
    @pl.when(pl.program_id(0) == 0)
    def _init():
        ua_ref[...] = jnp.zeros(ua_ref.shape, ua_ref.dtype)
        ub_ref[...] = jnp.zeros(ub_ref.shape, ub_ref.dtype)

    lane = jax.lax.broadcasted_iota(jnp.int32, (1, 128), 1)
    half = block_e // 2

    def one(j, ref):
        rw = rows_ref[0, 0, j]
        c = cols_ref[0, 0, j]
        v = vals_ref[0, 0, j]
        row = ent2_ref[pl.ds(c // 2, 1), :] * v
        sel = jnp.where(c % 2 == rw % 2, row, _swap_halves(row))
        contrib = jnp.where(lane // 64 == rw % 2, sel, 0.0)
        ref[pl.ds(rw // 2, 1), :] = ref[pl.ds(rw // 2, 1), :] + contrib

    def body(j, carry):
        one(j, ua_ref)
        one(j + half, ub_ref)
        return carry

    jax.lax.fori_loop(0, half, body, 0)


def _small_body(latent_ref, relemb_ref, att_ref, m_ref, cor_ref, *, temp):
    att = att_ref[...]
    mx = jnp.max(att, axis=-1, keepdims=True)
    e = jnp.exp(att - mx)
    sm = e / jnp.sum(e, axis=-1, keepdims=True)
    disen = jax.lax.dot_general(
        sm, relemb_ref[...], (((1,), (0,)), ((), ())),
        preferred_element_type=jnp.float32)
    m_ref[...] = jax.lax.dot_general(
        latent_ref[...], disen, (((0,), (0,)), ((), ())),
        preferred_element_type=jnp.float32)
    nrm = jnp.sqrt(jnp.sum(att * att, axis=-1, keepdims=True))
    natt = att / jnp.maximum(nrm, 1e-12)
    logits = jax.lax.dot_general(
        natt, natt, (((1,), (1,)), ((), ())),
        preferred_element_type=jnp.float32) / temp
    lmx = jnp.max(logits, axis=-1)
    lse = lmx + jnp.log(jnp.sum(jnp.exp(logits - lmx[:, None]), axis=-1))
    f = logits.shape[0]
    eye = (jax.lax.broadcasted_iota(jnp.int32, (f, f), 0) ==
           jax.lax.broadcasted_iota(jnp.int32, (f, f), 1))
    trace = jnp.sum(jnp.where(eye, logits, 0.0))
    cor_ref[...] = jnp.reshape(jnp.sum(lse) - trace, (1, 1))


def _ent_epilogue_body(agg_ref, deg_ref, resin_ref, new_ref, resout_ref):
    x = agg_ref[...] / jnp.maximum(deg_ref[...], 1.0)
    n = jnp.sqrt(jnp.sum(x * x, axis=-1, keepdims=True))
    y = x / jnp.maximum(n, 1e-12)
    new_ref[...] = y
    resout_ref[...] = resin_ref[...] + y


def _user_epilogue_body(ua_ref, ub_ref, usr_ref, m_ref, resin_ref, new_ref,
                        resout_ref):
    u = ua_ref[...] + ub_ref[...]
    factor = jax.lax.dot_general(
        usr_ref[...], m_ref[...], (((1,), (0,)), ((), ())),
        preferred_element_type=jnp.float32)
    x = u * factor + u
    n = jnp.sqrt(jnp.sum(x * x, axis=-1, keepdims=True))
    y = x / jnp.maximum(n, 1e-12)
    new_ref[...] = y
    resout_ref[...] = resin_ref[...] + y


def kernel(edge_index, edge_rel, im_rows, im_cols, im_vals, user_emb,
           entity_emb, latent_emb, rel_emb, disen_weight_att):
    num_edges = edge_rel.shape[0]
    nnz = im_rows.shape[0]
    num_users, dim = user_emb.shape
    num_entities = entity_emb.shape[0]
    temp = 0.2

    eb = _pick_block(num_edges, 4000)
    ge = num_edges // eb
    ib = _pick_block(nnz, 4000)
    gi = nnz // ib

    src3 = edge_index[0].reshape(ge, 1, eb)
    dst3 = edge_index[1].reshape(ge, 1, eb)
    rel3 = edge_rel.reshape(ge, 1, eb)
    rows3 = im_rows.reshape(gi, 1, ib)
    cols3 = im_cols.reshape(gi, 1, ib)
    vals3 = im_vals.reshape(gi, 1, ib)

    ent_pack = num_entities // 2
    usr_pack = num_users // 2
    deg_rows = -(-num_entities // 128)
    rel_dup = jnp.concatenate([rel_emb, rel_emb], axis=1)

    idx_spec = lambda b: pl.BlockSpec((1, 1, b), lambda i: (i, 0, 0),
                                      memory_space=pltpu.SMEM)
    full = lambda shape: pl.BlockSpec(shape, lambda i: tuple(0 for _ in shape))

    edge_call = pl.pallas_call(
        functools.partial(_edge_agg_body, block_e=eb),
        grid=(ge,),
        in_specs=[idx_spec(eb), idx_spec(eb), idx_spec(eb),
                  full((ent_pack, 128)), full(rel_dup.shape)],
        out_specs=[full((ent_pack, 128))],
        out_shape=[jax.ShapeDtypeStruct((ent_pack, 128), jnp.float32)],
    )

    deg_call = pl.pallas_call(
        functools.partial(_deg_body, block_e=eb),
        grid=(ge,),
        in_specs=[idx_spec(eb)],
        out_specs=[full((deg_rows, 128)), full((deg_rows, 128))],
        out_shape=[jax.ShapeDtypeStruct((deg_rows, 128), jnp.float32),
                   jax.ShapeDtypeStruct((deg_rows, 128), jnp.float32)],
    )

    user_call = pl.pallas_call(
        functools.partial(_user_agg_body, block_e=ib),
        grid=(gi,),
        in_specs=[idx_spec(ib), idx_spec(ib), idx_spec(ib),
                  full((ent_pack, 128))],
        out_specs=[full((usr_pack, 128)), full((usr_pack, 128))],
        out_shape=[jax.ShapeDtypeStruct((usr_pack, 128), jnp.float32),
                   jax.ShapeDtypeStruct((usr_pack, 128), jnp.float32)],
    )

    m_mat, cor2 = pl.pallas_call(
        functools.partial(_small_body, temp=temp),
        in_specs=[pl.BlockSpec(latent_emb.shape, lambda: (0, 0)),
                  pl.BlockSpec(rel_emb.shape, lambda: (0, 0)),
                  pl.BlockSpec(disen_weight_att.shape, lambda: (0, 0))],
        out_specs=[pl.BlockSpec((dim, dim), lambda: (0, 0)),
                   pl.BlockSpec((1, 1), lambda: (0, 0))],
        out_shape=[jax.ShapeDtypeStruct((dim, dim), jnp.float32),
                   jax.ShapeDtypeStruct((1, 1), jnp.float32)],
    )(latent_emb, rel_emb, disen_weight_att)

    entb = _pick_block(num_entities, 2000)
    gent = num_entities // entb
    ent_epi = pl.pallas_call(
        _ent_epilogue_body,
        grid=(gent,),
        in_specs=[pl.BlockSpec((entb, dim), lambda i: (i, 0)),
                  pl.BlockSpec((entb, 1), lambda i: (i, 0)),
                  pl.BlockSpec((entb, dim), lambda i: (i, 0))],
        out_specs=[pl.BlockSpec((entb, dim), lambda i: (i, 0)),
                   pl.BlockSpec((entb, dim), lambda i: (i, 0))],
        out_shape=[jax.ShapeDtypeStruct((num_entities, dim), jnp.float32),
                   jax.ShapeDtypeStruct((num_entities, dim), jnp.float32)],
    )

    usrb = _pick_block(num_users, 2000)
    gusr = num_users // usrb
    usr_epi = pl.pallas_call(
        _user_epilogue_body,
        grid=(gusr,),
        in_specs=[pl.BlockSpec((usrb, dim), lambda i: (i, 0)),
                  pl.BlockSpec((usrb, dim), lambda i: (i, 0)),
                  pl.BlockSpec((usrb, dim), lambda i: (i, 0)),
                  pl.BlockSpec((dim, dim), lambda i: (0, 0)),
                  pl.BlockSpec((usrb, dim), lambda i: (i, 0))],
        out_specs=[pl.BlockSpec((usrb, dim), lambda i: (i, 0)),
                   pl.BlockSpec((usrb, dim), lambda i: (i, 0))],
        out_shape=[jax.ShapeDtypeStruct((num_users, dim), jnp.float32),
                   jax.ShapeDtypeStruct((num_users, dim), jnp.float32)],
    )

    dega2, degb2 = deg_call(dst3)
    deg = (dega2 + degb2).reshape(deg_rows * 128)[:num_entities, None]

    ent = entity_emb
    usr = user_emb
    eres = entity_emb
    ures = user_emb
    for _ in range(2):
        ent2 = ent.reshape(ent_pack, 128)
        (agg2,) = edge_call(src3, dst3, rel3, ent2, rel_dup)
        agg = agg2.reshape(num_entities, dim)
        ua2, ub2 = user_call(rows3, cols3, vals3, ent2)
        ua = ua2.reshape(num_users, dim)
        ub = ub2.reshape(num_users, dim)
        ent, eres = ent_epi(agg, deg, eres)
        usr, ures = usr_epi(ua, ub, usr, m_mat, ures)

    return eres, ures, cor2.reshape(())
